# baseline (speedup 1.0000x reference)
import functools

import jax
import jax.numpy as jnp
from jax import lax
from jax.experimental import pallas as pl
from jax.experimental.pallas import tpu as pltpu
from jax.experimental.pallas import tpu_sc as plsc

BATCH = 256
N_OUT = BATCH // 2
LAM = 0.3
OML = 1.0 - LAM

IROWS = 3 * 224 * 224
NC = 2
NS = 16
NW = NC * NS
RPW = IROWS // NW
R = 112
NSTEPS = RPW // R
OG = N_OUT // 16

TROWS = 1024
TPW = TROWS // NW


def _mix_rows(pas, pbs, src, dst, nrows):
    @plsc.parallel_loop(0, nrows, 1, unroll=2)
    def _row(r):
        srow = ((r >> 3) << 11) + ((r & 7) << 7)
        for v in range(OG):
            a = plsc.load_gather(src, [pas[v] + srow])
            b = plsc.load_gather(src, [pbs[v] + srow])
            dst[pl.ds(r * N_OUT + v * 16, 16)] = a * LAM + b * OML


def _body(x, tgt, gidx, out_img, out_tgt, gv_v, abuf0, abuf1, abuf2,
          obuf0, obuf1, obuf2, si0, si1, si2, so0, so1, so2):
    wid = lax.axis_index("s") * NC + lax.axis_index("c")
    pltpu.sync_copy(gidx, gv_v)
    pas = []
    pbs = []
    for v in range(OG):
        pk = gv_v[0, pl.ds(v * 16, 16)]
        pas.append(pk & 0xFFFF)
        pbs.append(pk >> 16)

    tb = wid * TPW
    pltpu.sync_copy(tgt.at[pl.ds(tb * BATCH, TPW * BATCH)],
                    abuf0.at[pl.ds(0, TPW * BATCH)])
    _mix_rows(pas, pbs, abuf0, obuf0, TPW)
    pltpu.sync_copy(obuf0.at[pl.ds(0, TPW * N_OUT)],
                    out_tgt.at[pl.ds(tb * N_OUT, TPW * N_OUT)])

    wbase = wid * RPW
    abufs, obufs, sis, sos = ((abuf0, abuf1, abuf2), (obuf0, obuf1, obuf2),
                              (si0, si1, si2), (so0, so1, so2))

    for k in range(3):
        pltpu.async_copy(x.at[pl.ds((wbase + k * R) * BATCH, R * BATCH)],
                         abufs[k], sis[k])

    def _pair(tt, carry):
        for k in range(3):
            s = tt * 3 + k
            base = wbase + s * R
            pltpu.make_async_copy(x.at[pl.ds(base * BATCH, R * BATCH)],
                                  abufs[k], sis[k]).wait()

            @pl.when(tt > 0)
            def _drain():
                pltpu.make_async_copy(obufs[k],
                                      out_img.at[pl.ds(base * N_OUT, R * N_OUT)],
                                      sos[k]).wait()

            _mix_rows(pas, pbs, abufs[k], obufs[k], R)
            pltpu.async_copy(obufs[k], out_img.at[pl.ds(base * N_OUT, R * N_OUT)],
                             sos[k])

            @pl.when(tt < NSTEPS // 3 - 1)
            def _prefetch():
                pltpu.async_copy(x.at[pl.ds((base + 3 * R) * BATCH, R * BATCH)],
                                 abufs[k], sis[k])
        return carry

    lax.fori_loop(0, NSTEPS // 3, _pair, 0)
    for k in range(3):
        last = wbase + (NSTEPS - 3 + k) * R
        pltpu.make_async_copy(obufs[k],
                              out_img.at[pl.ds(last * N_OUT, R * N_OUT)],
                              sos[k]).wait()


def _phys_flat(mat2d):
    rows = mat2d.shape[0]
    return (mat2d.reshape(rows // 8, 8, 2, 128)
            .transpose(0, 2, 1, 3).reshape(-1))


@jax.jit
def _mixup(batch_image, batch_target, batch_group):
    order = jnp.argsort(batch_group, stable=True)
    idx0 = order[:N_OUT]
    idx1 = order[N_OUT:]
    perm0 = jax.random.permutation(jax.random.key(1), N_OUT)
    perm1 = jax.random.permutation(jax.random.key(2), N_OUT)
    g0 = idx0[perm0].astype(jnp.int32)
    g1 = idx1[perm1].astype(jnp.int32)
    pa = ((g0 >> 7) << 10) + (g0 & 127)
    pb = ((g1 >> 7) << 10) + (g1 & 127)
    gidx = (pa | (pb << 16)).reshape(1, N_OUT)

    x_t = batch_image.transpose(1, 2, 3, 0).reshape(IROWS, BATCH)
    x1 = _phys_flat(x_t)
    tgt_t = batch_target.T
    tgt_pad = jnp.pad(tgt_t, ((0, TROWS - tgt_t.shape[0]), (0, 0)))
    tgt1 = _phys_flat(tgt_pad)

    mesh = plsc.VectorSubcoreMesh(core_axis_name="c", subcore_axis_name="s")
    out_img, out_tgt = functools.partial(
        pl.kernel,
        mesh=mesh,
        compiler_params=pltpu.CompilerParams(use_tc_tiling_on_sc=True,
                                             needs_layout_passes=False),
        out_type=(
            jax.ShapeDtypeStruct((IROWS * N_OUT,), jnp.float32),
            jax.ShapeDtypeStruct((TROWS * N_OUT,), jnp.float32),
        ),
        scratch_types=[
            pltpu.VMEM((1, N_OUT), jnp.int32),
            pltpu.VMEM((R * BATCH,), jnp.float32),
            pltpu.VMEM((R * BATCH,), jnp.float32),
            pltpu.VMEM((R * BATCH,), jnp.float32),
            pltpu.VMEM((R * N_OUT,), jnp.float32),
            pltpu.VMEM((R * N_OUT,), jnp.float32),
            pltpu.VMEM((R * N_OUT,), jnp.float32),
            pltpu.SemaphoreType.DMA,
            pltpu.SemaphoreType.DMA,
            pltpu.SemaphoreType.DMA,
            pltpu.SemaphoreType.DMA,
            pltpu.SemaphoreType.DMA,
            pltpu.SemaphoreType.DMA,
        ],
    )(_body)(x1, tgt1, gidx)

    inputs_mix = out_img.reshape(3, 224, 224, N_OUT).transpose(3, 0, 1, 2)
    targets_mix = out_tgt.reshape(TROWS, N_OUT)[: batch_target.shape[1]].T
    return inputs_mix, targets_mix


def kernel(batch_image, batch_target, batch_group):
    return _mixup(batch_image, batch_target, batch_group)


# speedup vs baseline: 1.0881x; 1.0881x over previous
import functools

import jax
import jax.numpy as jnp
from jax import lax
from jax.experimental import pallas as pl
from jax.experimental.pallas import tpu as pltpu
from jax.experimental.pallas import tpu_sc as plsc

BATCH = 256
N_OUT = BATCH // 2
LAM = 0.3
OML = 1.0 - LAM

IROWS = 3 * 224 * 224
NC = 2
NS = 16
NW = NC * NS
RPW = IROWS // NW
R = 112
NSTEPS = RPW // R
OG = N_OUT // 16

TROWS = 1024
TPW = TROWS // NW


def _mix_rows(pas, pbs, src, dst, nrows):
    @plsc.parallel_loop(0, nrows, 1, unroll=2)
    def _row(r):
        srow = ((r >> 3) << 11) + ((r & 7) << 7)
        for v in range(OG):
            a = plsc.load_gather(src, [pas[v] + srow])
            b = plsc.load_gather(src, [pbs[v] + srow])
            dst[pl.ds(r * N_OUT + v * 16, 16)] = a * LAM + b * OML


def _body(x, tgt, gidx, out_img, out_tgt, gv_v, abuf0, abuf1, obuf0, obuf1,
          tbuf, tobuf, si0, si1, so0, so1):
    wid = lax.axis_index("s") * NC + lax.axis_index("c")
    pltpu.sync_copy(gidx, gv_v)
    pas = []
    pbs = []
    for v in range(OG):
        ga = gv_v[0, pl.ds(v * 16, 16)]
        gb = gv_v[0, pl.ds(N_OUT + v * 16, 16)]
        pas.append(((ga >> 7) << 10) + (ga & 127))
        pbs.append(((gb >> 7) << 10) + (gb & 127))

    tb = wid * TPW
    pltpu.sync_copy(tgt.at[pl.ds(tb * BATCH, TPW * BATCH)], tbuf)
    _mix_rows(pas, pbs, tbuf, tobuf, TPW)
    pltpu.sync_copy(tobuf, out_tgt.at[pl.ds(tb * N_OUT, TPW * N_OUT)])

    wbase = wid * RPW
    abufs, obufs, sis, sos = (abuf0, abuf1), (obuf0, obuf1), (si0, si1), (so0, so1)

    for k in range(2):
        pltpu.async_copy(x.at[pl.ds((wbase + k * R) * BATCH, R * BATCH)],
                         abufs[k], sis[k])

    def _pair(tt, carry):
        for k in range(2):
            s = tt * 2 + k
            base = wbase + s * R
            pltpu.make_async_copy(x.at[pl.ds(base * BATCH, R * BATCH)],
                                  abufs[k], sis[k]).wait()

            @pl.when(tt > 0)
            def _drain():
                pltpu.make_async_copy(obufs[k],
                                      out_img.at[pl.ds(base * N_OUT, R * N_OUT)],
                                      sos[k]).wait()

            _mix_rows(pas, pbs, abufs[k], obufs[k], R)
            pltpu.async_copy(obufs[k], out_img.at[pl.ds(base * N_OUT, R * N_OUT)],
                             sos[k])

            @pl.when(tt < NSTEPS // 2 - 1)
            def _prefetch():
                pltpu.async_copy(x.at[pl.ds((base + 2 * R) * BATCH, R * BATCH)],
                                 abufs[k], sis[k])
        return carry

    lax.fori_loop(0, NSTEPS // 2, _pair, 0)
    for k in range(2):
        last = wbase + (NSTEPS - 2 + k) * R
        pltpu.make_async_copy(obufs[k],
                              out_img.at[pl.ds(last * N_OUT, R * N_OUT)],
                              sos[k]).wait()


def _phys_flat(mat2d):
    rows = mat2d.shape[0]
    return (mat2d.reshape(rows // 8, 8, 2, 128)
            .transpose(0, 2, 1, 3).reshape(-1))


@jax.jit
def _mixup(batch_image, batch_target, batch_group):
    order = jnp.argsort(batch_group, stable=True)
    idx0 = order[:N_OUT]
    idx1 = order[N_OUT:]
    perm0 = jax.random.permutation(jax.random.key(1), N_OUT)
    perm1 = jax.random.permutation(jax.random.key(2), N_OUT)
    g0 = idx0[perm0].astype(jnp.int32)
    g1 = idx1[perm1].astype(jnp.int32)
    gidx = jnp.concatenate([g0, g1]).reshape(1, 2 * N_OUT)

    x_t = batch_image.transpose(1, 2, 3, 0).reshape(IROWS, BATCH)
    x1 = _phys_flat(x_t)
    tgt_t = batch_target.T
    tgt_pad = jnp.pad(tgt_t, ((0, TROWS - tgt_t.shape[0]), (0, 0)))
    tgt1 = _phys_flat(tgt_pad)

    mesh = plsc.VectorSubcoreMesh(core_axis_name="c", subcore_axis_name="s")
    out_img, out_tgt = functools.partial(
        pl.kernel,
        mesh=mesh,
        compiler_params=pltpu.CompilerParams(use_tc_tiling_on_sc=True,
                                             needs_layout_passes=False),
        out_type=(
            jax.ShapeDtypeStruct((IROWS * N_OUT,), jnp.float32),
            jax.ShapeDtypeStruct((TROWS * N_OUT,), jnp.float32),
        ),
        scratch_types=[
            pltpu.VMEM((1, 2 * N_OUT), jnp.int32),
            pltpu.VMEM((R * BATCH,), jnp.float32),
            pltpu.VMEM((R * BATCH,), jnp.float32),
            pltpu.VMEM((R * N_OUT,), jnp.float32),
            pltpu.VMEM((R * N_OUT,), jnp.float32),
            pltpu.VMEM((TPW * BATCH,), jnp.float32),
            pltpu.VMEM((TPW * N_OUT,), jnp.float32),
            pltpu.SemaphoreType.DMA,
            pltpu.SemaphoreType.DMA,
            pltpu.SemaphoreType.DMA,
            pltpu.SemaphoreType.DMA,
        ],
    )(_body)(x1, tgt1, gidx)

    inputs_mix = out_img.reshape(3, 224, 224, N_OUT).transpose(3, 0, 1, 2)
    targets_mix = out_tgt.reshape(TROWS, N_OUT)[: batch_target.shape[1]].T
    return inputs_mix, targets_mix


def kernel(batch_image, batch_target, batch_group):
    return _mixup(batch_image, batch_target, batch_group)
